# baseline (speedup 1.0000x reference)
import functools

import jax
import jax.numpy as jnp
from jax import lax
from jax.experimental import pallas as pl
from jax.experimental.pallas import tpu as pltpu
from jax.experimental.pallas import tpu_sc as plsc

DIM = 128
HW = DIM // 2
NW = 32
R = 128


def _pack_table(tab):
    v = tab.shape[0]
    pairs = tab.reshape(v, DIM // 32, 2, 16).transpose(0, 1, 3, 2)
    return lax.bitcast_convert_type(
        pairs.astype(jnp.bfloat16), jnp.int32).reshape(v, HW)


def _build(n_rows):
    per_w = n_rows // NW
    steps = per_w // R
    assert steps % 2 == 0
    mesh = plsc.VectorSubcoreMesh(core_axis_name="c", subcore_axis_name="s")

    @functools.partial(
        pl.kernel,
        mesh=mesh,
        compiler_params=pltpu.CompilerParams(
            needs_layout_passes=False, use_tc_tiling_on_sc=False),
        out_type=jax.ShapeDtypeStruct((n_rows, DIM), jnp.float32),
        scratch_types=[
            pltpu.VMEM((steps, R), jnp.int32),
            pltpu.VMEM((steps, R), jnp.int32),
            pltpu.VMEM((R, DIM), jnp.float32),
            pltpu.VMEM((R, DIM), jnp.float32),
            pltpu.VMEM((R, HW), jnp.int32),
            pltpu.VMEM((R, HW), jnp.int32),
            pltpu.SemaphoreType.DMA,
            pltpu.SemaphoreType.DMA,
            pltpu.SemaphoreType.DMA,
            pltpu.SemaphoreType.DMA,
        ],
    )
    def k(tok_idx_hbm, pos_idx_hbm, emb_hbm, pos_hbm, out_hbm,
          idx_tok, idx_pos, tok0, tok1, pos0, pos1,
          sem_g0, sem_g1, sem_o0, sem_o1):
        wid = lax.axis_index("s") * 2 + lax.axis_index("c")
        wbase = wid * per_w
        tok_b = (tok0, tok1)
        pos_b = (pos0, pos1)
        sem_g = (sem_g0, sem_g1)
        sem_o = (sem_o0, sem_o1)

        pltpu.sync_copy(tok_idx_hbm.at[wid], idx_tok)
        pltpu.sync_copy(pos_idx_hbm.at[wid], idx_pos)

        def issue(si, p):
            pltpu.async_copy(emb_hbm.at[idx_tok.at[si]], tok_b[p], sem_g[p])
            pltpu.async_copy(pos_hbm.at[idx_pos.at[si]], pos_b[p], sem_g[p])

        def wait_gathers(si, p):
            pltpu.make_async_copy(emb_hbm.at[idx_tok.at[si]], tok_b[p], sem_g[p]).wait()
            pltpu.make_async_copy(pos_hbm.at[idx_pos.at[si]], pos_b[p], sem_g[p]).wait()

        def add(p):
            tb, pb = tok_b[p], pos_b[p]
            himask = jnp.int32(-65536)

            def lo(w):
                return plsc.bitcast(lax.shift_left(w, 16), jnp.float32)

            def hi(w):
                return plsc.bitcast(w & himask, jnp.float32)

            def add_rows(r2, c):
                for dr in range(2):
                    r = r2 * 2 + dr
                    for jj in range(DIM // 32):
                        wp = pb[r, pl.ds(jj * 16, 16)]
                        sa = pl.ds(jj * 32, 16)
                        sb = pl.ds(jj * 32 + 16, 16)
                        tb[r, sa] = tb[r, sa] + lo(wp)
                        tb[r, sb] = tb[r, sb] + hi(wp)
                return c

            lax.fori_loop(0, R // 2, add_rows, 0)

        def start_out(si, p):
            pltpu.async_copy(tok_b[p], out_hbm.at[pl.ds(wbase + si * R, R)], sem_o[p])

        def wait_out(si, p):
            pltpu.make_async_copy(
                tok_b[p], out_hbm.at[pl.ds(wbase + si * R, R)], sem_o[p]).wait()

        issue(0, 0)

        def body(i2, carry):
            i0 = i2 * 2
            i1 = i0 + 1

            @pl.when(i2 > 0)
            def _():
                wait_out(i0 - 1, 1)

            issue(i1, 1)
            wait_gathers(i0, 0)
            add(0)
            start_out(i0, 0)

            @pl.when(i2 < steps // 2 - 1)
            def _():
                wait_out(i0, 0)
                issue(i0 + 2, 0)

            wait_gathers(i1, 1)
            add(1)
            start_out(i1, 1)
            return carry

        lax.fori_loop(0, steps // 2, body, 0)
        wait_out(steps - 2, 0)
        wait_out(steps - 1, 1)

    return k


@jax.jit
def kernel(input_tensor, incremental_mask, emb_table, pos_table):
    b, l = input_tensor.shape
    n = b * l
    per_w = n // NW
    steps = per_w // R
    tok_idx = input_tensor.reshape(NW, steps, R)
    pos_idx = incremental_mask.reshape(NW, steps, R)
    out = _build(n)(tok_idx, pos_idx, emb_table, _pack_table(pos_table))
    return out.reshape(b, l, DIM)


# speedup vs baseline: 1.2024x; 1.2024x over previous
import functools

import jax
import jax.numpy as jnp
from jax import lax
from jax.experimental import pallas as pl
from jax.experimental.pallas import tpu as pltpu
from jax.experimental.pallas import tpu_sc as plsc

DIM = 128
HW = DIM // 2
NW = 32
R = 128


def _pack_table(tab):
    v = tab.shape[0]
    pairs = tab.reshape(v, DIM // 32, 2, 16).transpose(0, 1, 3, 2)
    return lax.bitcast_convert_type(
        pairs.astype(jnp.bfloat16), jnp.int32).reshape(v, HW)


def _build(n_rows):
    per_w = n_rows // NW
    steps = per_w // R
    assert steps % 4 == 0
    mesh = plsc.VectorSubcoreMesh(core_axis_name="c", subcore_axis_name="s")

    @functools.partial(
        pl.kernel,
        mesh=mesh,
        compiler_params=pltpu.CompilerParams(
            needs_layout_passes=False, use_tc_tiling_on_sc=False),
        out_type=jax.ShapeDtypeStruct((n_rows, DIM), jnp.float32),
        scratch_types=[
            pltpu.VMEM((steps, R), jnp.int32),
            pltpu.VMEM((steps, R), jnp.int32),
            [pltpu.VMEM((R, DIM), jnp.float32) for _ in range(4)],
            [pltpu.VMEM((R, HW), jnp.int32) for _ in range(2)],
            [pltpu.SemaphoreType.DMA for _ in range(4)],
            [pltpu.SemaphoreType.DMA for _ in range(2)],
            [pltpu.SemaphoreType.DMA for _ in range(4)],
        ],
    )
    def k(tok_idx_hbm, pos_idx_hbm, emb_hbm, pos_hbm, out_hbm,
          idx_tok, idx_pos, tok_b, pos_b, sem_t, sem_p, sem_o):
        wid = lax.axis_index("s") * 2 + lax.axis_index("c")
        wbase = wid * per_w

        pltpu.sync_copy(tok_idx_hbm.at[wid], idx_tok)
        pltpu.sync_copy(pos_idx_hbm.at[wid], idx_pos)

        def issue_tok(si, kn):
            pltpu.async_copy(
                emb_hbm.at[idx_tok.at[si]], tok_b[kn % 4], sem_t[kn % 4])

        def wait_tok(si, kn):
            pltpu.make_async_copy(
                emb_hbm.at[idx_tok.at[si]], tok_b[kn % 4], sem_t[kn % 4]).wait()

        def issue_pos(si, kn):
            pltpu.async_copy(
                pos_hbm.at[idx_pos.at[si]], pos_b[kn % 2], sem_p[kn % 2])

        def wait_pos(si, kn):
            pltpu.make_async_copy(
                pos_hbm.at[idx_pos.at[si]], pos_b[kn % 2], sem_p[kn % 2]).wait()

        def add(kn):
            tb, pb = tok_b[kn % 4], pos_b[kn % 2]
            himask = jnp.int32(-65536)

            def lo(w):
                return plsc.bitcast(lax.shift_left(w, 16), jnp.float32)

            def hi(w):
                return plsc.bitcast(w & himask, jnp.float32)

            def add_row(r, c):
                for jj in range(DIM // 32):
                    wp = pb[r, pl.ds(jj * 16, 16)]
                    sa = pl.ds(jj * 32, 16)
                    sb = pl.ds(jj * 32 + 16, 16)
                    tb[r, sa] = tb[r, sa] + lo(wp)
                    tb[r, sb] = tb[r, sb] + hi(wp)
                return c

            lax.fori_loop(0, R, add_row, 0)

        def start_out(si, kn):
            pltpu.async_copy(
                tok_b[kn % 4], out_hbm.at[pl.ds(wbase + si * R, R)], sem_o[kn % 4])

        def wait_out(si, kn):
            pltpu.make_async_copy(
                tok_b[kn % 4], out_hbm.at[pl.ds(wbase + si * R, R)],
                sem_o[kn % 4]).wait()

        def half(si, kn):
            @pl.when(si + 2 < steps)
            def _():
                @pl.when(si >= 2)
                def _():
                    wait_out(si - 2, kn + 2)

                issue_tok(si + 2, kn + 2)

            @pl.when(si + 1 < steps)
            def _():
                issue_pos(si + 1, kn + 1)

            wait_tok(si, kn)
            wait_pos(si, kn)
            add(kn)
            start_out(si, kn)

        issue_tok(0, 0)
        issue_tok(1, 1)
        issue_pos(0, 0)

        def body(i4, carry):
            base = i4 * 4
            for kk in range(4):
                half(base + kk, kk)
            return carry

        lax.fori_loop(0, steps // 4, body, 0)
        wait_out(steps - 2, steps - 2)
        wait_out(steps - 1, steps - 1)

    return k


@jax.jit
def kernel(input_tensor, incremental_mask, emb_table, pos_table):
    b, l = input_tensor.shape
    n = b * l
    per_w = n // NW
    steps = per_w // R
    tok_idx = input_tensor.reshape(NW, steps, R)
    pos_idx = incremental_mask.reshape(NW, steps, R)
    out = _build(n)(tok_idx, pos_idx, emb_table, _pack_table(pos_table))
    return out.reshape(b, l, DIM)
